# baseline (speedup 1.0000x reference)
import functools

import jax
import jax.numpy as jnp
from jax import lax
from jax.experimental import pallas as pl
from jax.experimental.pallas import tpu as pltpu
from jax.experimental.pallas import tpu_sc as plsc

_N = 1024
_L = 16

_mesh = plsc.VectorSubcoreMesh(core_axis_name="c", subcore_axis_name="s")


_GATHER_DNUMS = lax.GatherDimensionNumbers(
    offset_dims=(), collapsed_slice_dims=(0,), start_index_map=(0,)
)


def _lane_perm(v, idx):
    return lax.gather(
        v,
        idx[:, None],
        _GATHER_DNUMS,
        slice_sizes=(1,),
        mode=lax.GatherScatterMode.PROMISE_IN_BOUNDS,
    )


def _lane_allsum(v):
    idx = lax.iota(jnp.int32, _L)
    for s in (1, 2, 4, 8):
        v = v + _lane_perm(v, idx ^ s)
    return v


@functools.partial(
    pl.kernel,
    mesh=_mesh,
    out_type=jax.ShapeDtypeStruct((_L,), jnp.float32),
    scratch_types=[
        pltpu.VMEM((_N,), jnp.float32),
        pltpu.VMEM((_N,), jnp.int32),
        pltpu.VMEM((_L,), jnp.float32),
    ],
)
def _auroc_sc(pred_hbm, true_hbm, out_hbm, pred_v, true_v, out_v):
    cid = lax.axis_index("c")
    sid = lax.axis_index("s")

    @pl.when(jnp.logical_and(cid == 0, sid == 0))
    def _():
        pltpu.sync_copy(pred_hbm, pred_v)
        pltpu.sync_copy(true_hbm, true_v)

        one = jnp.float32(1.0)
        zero = jnp.zeros((_L,), jnp.float32)
        sum_pos = zero
        sum_all = zero
        cnt_pos = zero
        for i in range(_N // _L):
            x = pred_v[pl.ds(i * _L, _L)]
            t = true_v[pl.ds(i * _L, _L)]
            posf = t.astype(jnp.float32)
            p = one / (one + jnp.exp(-x))
            sum_pos = sum_pos + p * posf
            sum_all = sum_all + p
            cnt_pos = cnt_pos + posf

        sp = _lane_allsum(sum_pos)
        sn = _lane_allsum(sum_all) - sp
        P = _lane_allsum(cnt_pos)
        Nn = jnp.float32(_N) - P
        denom = P * Nn
        out_v[...] = (denom - Nn * sp + P * sn) / jnp.maximum(denom, one)
        pltpu.sync_copy(out_v, out_hbm)


def kernel(y_pred, y_true):
    out = _auroc_sc(y_pred.astype(jnp.float32), y_true.astype(jnp.int32))
    return out[0]


# speedup vs baseline: 1.0652x; 1.0652x over previous
import functools

import jax
import jax.numpy as jnp
from jax import lax
from jax.experimental import pallas as pl
from jax.experimental.pallas import tpu as pltpu
from jax.experimental.pallas import tpu_sc as plsc

_N = 1024
_L = 16
_NS = 16
_PER = _N // _NS
_PW = 3 * _L

_mesh = plsc.VectorSubcoreMesh(core_axis_name="c", subcore_axis_name="s")

_GATHER_DNUMS = lax.GatherDimensionNumbers(
    offset_dims=(), collapsed_slice_dims=(0,), start_index_map=(0,)
)


def _lane_perm(v, idx):
    return lax.gather(
        v,
        idx[:, None],
        _GATHER_DNUMS,
        slice_sizes=(1,),
        mode=lax.GatherScatterMode.PROMISE_IN_BOUNDS,
    )


def _lane_allsum(v):
    idx = lax.iota(jnp.int32, _L)
    for s in (1, 2, 4, 8):
        v = v + _lane_perm(v, idx ^ s)
    return v


@functools.partial(
    pl.kernel,
    mesh=_mesh,
    out_type=jax.ShapeDtypeStruct((_L,), jnp.float32),
    scratch_types=[
        pltpu.VMEM((_PER,), jnp.float32),
        pltpu.VMEM((_PER,), jnp.int32),
        pltpu.VMEM((_PW,), jnp.float32),
        pltpu.VMEM((_NS * _PW,), jnp.float32),
        pltpu.VMEM_SHARED((_NS * _PW,), jnp.float32),
        pltpu.VMEM((_L,), jnp.float32),
    ],
)
def _auroc_sc(pred_hbm, true_hbm, out_hbm, pred_v, true_v, part_v, acc_v, shared, out_v):
    cid = lax.axis_index("c")
    sid = lax.axis_index("s")

    @pl.when(cid == 0)
    def _():
        base = sid * _PER
        pltpu.sync_copy(pred_hbm.at[pl.ds(base, _PER)], pred_v)
        pltpu.sync_copy(true_hbm.at[pl.ds(base, _PER)], true_v)

        one = jnp.float32(1.0)
        zero = jnp.zeros((_L,), jnp.float32)
        sum_pos = zero
        sum_all = zero
        cnt_pos = zero
        for i in range(_PER // _L):
            x = pred_v[pl.ds(i * _L, _L)]
            t = true_v[pl.ds(i * _L, _L)]
            posf = t.astype(jnp.float32)
            p = one / (one + jnp.exp(-x))
            sum_pos = sum_pos + p * posf
            sum_all = sum_all + p
            cnt_pos = cnt_pos + posf

        part_v[pl.ds(0, _L)] = sum_pos
        part_v[pl.ds(_L, _L)] = sum_all
        part_v[pl.ds(2 * _L, _L)] = cnt_pos
        pltpu.sync_copy(part_v, shared.at[pl.ds(sid * _PW, _PW)])
        plsc.subcore_barrier()

        @pl.when(sid == 0)
        def _():
            pltpu.sync_copy(shared, acc_v)
            sp = jnp.zeros((_L,), jnp.float32)
            sa = jnp.zeros((_L,), jnp.float32)
            cp = jnp.zeros((_L,), jnp.float32)
            for r in range(_NS):
                sp = sp + acc_v[pl.ds(r * _PW, _L)]
                sa = sa + acc_v[pl.ds(r * _PW + _L, _L)]
                cp = cp + acc_v[pl.ds(r * _PW + 2 * _L, _L)]
            sp = _lane_allsum(sp)
            sn = _lane_allsum(sa) - sp
            P = _lane_allsum(cp)
            Nn = jnp.float32(_N) - P
            denom = P * Nn
            out_v[...] = (denom - Nn * sp + P * sn) / jnp.maximum(denom, one)
            pltpu.sync_copy(out_v, out_hbm)


def kernel(y_pred, y_true):
    out = _auroc_sc(y_pred.astype(jnp.float32), y_true.astype(jnp.int32))
    return out[0]


# speedup vs baseline: 1.0923x; 1.0254x over previous
import functools

import jax
import jax.numpy as jnp
from jax import lax
from jax.experimental import pallas as pl
from jax.experimental.pallas import tpu as pltpu
from jax.experimental.pallas import tpu_sc as plsc

_N = 1024
_L = 16
_NS = 16
_PER = _N // _NS
_PW = 3 * _L

_mesh = plsc.VectorSubcoreMesh(core_axis_name="c", subcore_axis_name="s")

_GATHER_DNUMS = lax.GatherDimensionNumbers(
    offset_dims=(), collapsed_slice_dims=(0,), start_index_map=(0,)
)


def _lane_perm(v, idx):
    return lax.gather(
        v,
        idx[:, None],
        _GATHER_DNUMS,
        slice_sizes=(1,),
        mode=lax.GatherScatterMode.PROMISE_IN_BOUNDS,
    )


def _lane_allsum(v):
    idx = lax.iota(jnp.int32, _L)
    for s in (1, 2, 4, 8):
        v = v + _lane_perm(v, idx ^ s)
    return v


@functools.partial(
    pl.kernel,
    mesh=_mesh,
    out_type=jax.ShapeDtypeStruct((_L,), jnp.float32),
    scratch_types=[
        pltpu.VMEM((_PER,), jnp.float32),
        pltpu.VMEM((_PER,), jnp.int32),
        pltpu.VMEM((_PW,), jnp.float32),
        pltpu.VMEM((_NS * _PW,), jnp.float32),
        pltpu.VMEM_SHARED((_NS * _PW,), jnp.float32),
        pltpu.VMEM((_L,), jnp.float32),
        pltpu.SemaphoreType.DMA,
        pltpu.SemaphoreType.DMA,
    ],
)
def _auroc_sc(
    pred_hbm, true_hbm, out_hbm, pred_v, true_v, part_v, acc_v, shared, out_v,
    sem_a, sem_b,
):
    cid = lax.axis_index("c")
    sid = lax.axis_index("s")

    @pl.when(cid == 0)
    def _():
        base = sid * _PER
        cp_a = pltpu.async_copy(pred_hbm.at[pl.ds(base, _PER)], pred_v, sem_a)
        cp_b = pltpu.async_copy(true_hbm.at[pl.ds(base, _PER)], true_v, sem_b)
        cp_a.wait()
        cp_b.wait()

        one = jnp.float32(1.0)
        zero = jnp.zeros((_L,), jnp.float32)
        sum_pos = zero
        sum_all = zero
        cnt_pos = zero
        for i in range(_PER // _L):
            x = pred_v[pl.ds(i * _L, _L)]
            t = true_v[pl.ds(i * _L, _L)]
            posf = t.astype(jnp.float32)
            p = one / (one + jnp.exp(-x))
            sum_pos = sum_pos + p * posf
            sum_all = sum_all + p
            cnt_pos = cnt_pos + posf

        part_v[pl.ds(0, _L)] = sum_pos
        part_v[pl.ds(_L, _L)] = sum_all
        part_v[pl.ds(2 * _L, _L)] = cnt_pos
        pltpu.sync_copy(part_v, shared.at[pl.ds(sid * _PW, _PW)])
        plsc.subcore_barrier()

        @pl.when(sid == 0)
        def _():
            pltpu.sync_copy(shared, acc_v)
            sp = jnp.zeros((_L,), jnp.float32)
            sa = jnp.zeros((_L,), jnp.float32)
            cp = jnp.zeros((_L,), jnp.float32)
            for r in range(_NS):
                sp = sp + acc_v[pl.ds(r * _PW, _L)]
                sa = sa + acc_v[pl.ds(r * _PW + _L, _L)]
                cp = cp + acc_v[pl.ds(r * _PW + 2 * _L, _L)]
            sp = _lane_allsum(sp)
            sn = _lane_allsum(sa) - sp
            P = _lane_allsum(cp)
            Nn = jnp.float32(_N) - P
            denom = P * Nn
            out_v[...] = (denom - Nn * sp + P * sn) / jnp.maximum(denom, one)
            pltpu.sync_copy(out_v, out_hbm)


def kernel(y_pred, y_true):
    out = _auroc_sc(y_pred.astype(jnp.float32), y_true.astype(jnp.int32))
    return out[0]
